# baseline (speedup 1.0000x reference)
import jax
import jax.numpy as jnp
from jax.experimental import pallas as pl
from jax.experimental.pallas import tpu as pltpu

_BM = 1024
_NBUF = 4
_OCHUNK = 4
_M = 32768
_STEPS = _M // _BM
_NCHUNK = _STEPS // _OCHUNK


def _gate_kernel(x_hbm, w_ref, out_hbm, xbuf, obuf, insem, outsem):
    w = w_ref[...]

    def in_copy(i, slot):
        return pltpu.make_async_copy(
            x_hbm.at[pl.ds(i * _BM, _BM), :], xbuf.at[slot], insem.at[slot]
        )

    def out_copy(c, oslot):
        return pltpu.make_async_copy(
            obuf.at[oslot],
            out_hbm.at[:, pl.ds(c * _BM * _OCHUNK, _BM * _OCHUNK)],
            outsem.at[oslot],
        )

    for i in range(_NBUF - 1):
        in_copy(i, i).start()
    for i in range(_STEPS):
        slot = i % _NBUF
        c, j = divmod(i, _OCHUNK)
        oslot = c % 2
        in_copy(i, slot).wait()
        nxt = i + _NBUF - 1
        if nxt < _STEPS:
            in_copy(nxt, nxt % _NBUF).start()
        if j == 0 and c >= 2:
            out_copy(c - 2, oslot).wait()
        obuf[oslot, :, j * _BM:(j + 1) * _BM] = jax.lax.dot_general(
            w, xbuf[slot],
            dimension_numbers=(((1,), (1,)), ((), ())),
            preferred_element_type=jnp.float32,
        )
        if j == _OCHUNK - 1:
            out_copy(c, oslot).start()
    for c in (_NCHUNK - 2, _NCHUNK - 1):
        out_copy(c, c % 2).wait()


def kernel(x, W):
    m, d = x.shape
    e = W.shape[0]
    logits_t = pl.pallas_call(
        _gate_kernel,
        in_specs=[
            pl.BlockSpec(memory_space=pltpu.MemorySpace.HBM),
            pl.BlockSpec(memory_space=pltpu.MemorySpace.VMEM),
        ],
        out_specs=pl.BlockSpec(memory_space=pltpu.MemorySpace.HBM),
        out_shape=jax.ShapeDtypeStruct((e, m), jnp.float32),
        scratch_shapes=[
            pltpu.VMEM((_NBUF, _BM, d), jnp.float32),
            pltpu.VMEM((2, e, _BM * _OCHUNK), jnp.float32),
            pltpu.SemaphoreType.DMA((_NBUF,)),
            pltpu.SemaphoreType.DMA((2,)),
        ],
    )(x, W)
    return (logits_t.T, 2)


# speedup vs baseline: 1.0058x; 1.0058x over previous
import jax
import jax.numpy as jnp
from jax.experimental import pallas as pl
from jax.experimental.pallas import tpu as pltpu

_BM = 1024
_NBUF = 4
_OCHUNK = 8
_M = 32768
_STEPS = _M // _BM
_NCHUNK = _STEPS // _OCHUNK


def _gate_kernel(x_hbm, w_ref, out_hbm, xbuf, obuf, insem, outsem):
    w = w_ref[...]

    def in_copy(i, slot):
        return pltpu.make_async_copy(
            x_hbm.at[pl.ds(i * _BM, _BM), :], xbuf.at[slot], insem.at[slot]
        )

    def out_copy(c, oslot):
        return pltpu.make_async_copy(
            obuf.at[oslot],
            out_hbm.at[:, pl.ds(c * _BM * _OCHUNK, _BM * _OCHUNK)],
            outsem.at[oslot],
        )

    for i in range(_NBUF - 1):
        in_copy(i, i).start()
    for i in range(_STEPS):
        slot = i % _NBUF
        c, j = divmod(i, _OCHUNK)
        oslot = c % 2
        in_copy(i, slot).wait()
        nxt = i + _NBUF - 1
        if nxt < _STEPS:
            in_copy(nxt, nxt % _NBUF).start()
        if j == 0 and c >= 2:
            out_copy(c - 2, oslot).wait()
        obuf[oslot, :, j * _BM:(j + 1) * _BM] = jax.lax.dot_general(
            w, xbuf[slot],
            dimension_numbers=(((1,), (1,)), ((), ())),
            preferred_element_type=jnp.float32,
        )
        if j == _OCHUNK - 1:
            out_copy(c, oslot).start()
    for c in (_NCHUNK - 2, _NCHUNK - 1):
        out_copy(c, c % 2).wait()


def kernel(x, W):
    m, d = x.shape
    e = W.shape[0]
    logits_t = pl.pallas_call(
        _gate_kernel,
        in_specs=[
            pl.BlockSpec(memory_space=pltpu.MemorySpace.HBM),
            pl.BlockSpec(memory_space=pltpu.MemorySpace.VMEM),
        ],
        out_specs=pl.BlockSpec(memory_space=pltpu.MemorySpace.HBM),
        out_shape=jax.ShapeDtypeStruct((e, m), jnp.float32),
        scratch_shapes=[
            pltpu.VMEM((_NBUF, _BM, d), jnp.float32),
            pltpu.VMEM((2, e, _BM * _OCHUNK), jnp.float32),
            pltpu.SemaphoreType.DMA((_NBUF,)),
            pltpu.SemaphoreType.DMA((2,)),
        ],
    )(x, W)
    return (logits_t.T, 2)


# speedup vs baseline: 1.0163x; 1.0105x over previous
import jax
import jax.numpy as jnp
from jax.experimental import pallas as pl
from jax.experimental.pallas import tpu as pltpu

_BM = 1024
_NBUF = 4
_OCHUNK = 4
_M = 32768
_STEPS = _M // _BM
_NCHUNK = _STEPS // _OCHUNK


def _gate_kernel(x_hbm, w_ref, out_hbm, xbuf, obuf, insem, outsem):
    w = w_ref[...]

    def in_copy(i, slot):
        return pltpu.make_async_copy(
            x_hbm.at[pl.ds(i * _BM, _BM), :], xbuf.at[slot], insem.at[slot]
        )

    def out_copy(c, oslot):
        return pltpu.make_async_copy(
            obuf.at[oslot],
            out_hbm.at[:, pl.ds(c * _BM * _OCHUNK, _BM * _OCHUNK)],
            outsem.at[oslot],
        )

    for i in range(_NBUF - 1):
        in_copy(i, i).start()
    for i in range(_STEPS):
        slot = i % _NBUF
        c, j = divmod(i, _OCHUNK)
        oslot = c % 2
        in_copy(i, slot).wait()
        nxt = i + _NBUF - 1
        if nxt < _STEPS:
            in_copy(nxt, nxt % _NBUF).start()
        if j == 0 and c >= 2:
            out_copy(c - 2, oslot).wait()
        obuf[oslot, :, j * _BM:(j + 1) * _BM] = jax.lax.dot_general(
            w, xbuf[slot],
            dimension_numbers=(((1,), (1,)), ((), ())),
            preferred_element_type=jnp.float32,
        )
        if j == _OCHUNK - 1:
            out_copy(c, oslot).start()
    for c in (_NCHUNK - 2, _NCHUNK - 1):
        out_copy(c, c % 2).wait()


def kernel(x, W):
    m, d = x.shape
    e = W.shape[0]
    logits_t = pl.pallas_call(
        _gate_kernel,
        in_specs=[
            pl.BlockSpec(memory_space=pltpu.MemorySpace.HBM),
            pl.BlockSpec(memory_space=pltpu.MemorySpace.VMEM),
        ],
        out_specs=pl.BlockSpec(memory_space=pltpu.MemorySpace.HBM),
        out_shape=jax.ShapeDtypeStruct((e, m), jnp.float32),
        scratch_shapes=[
            pltpu.VMEM((_NBUF, _BM, d), jnp.float32),
            pltpu.VMEM((2, e, _BM * _OCHUNK), jnp.float32),
            pltpu.SemaphoreType.DMA((_NBUF,)),
            pltpu.SemaphoreType.DMA((2,)),
        ],
    )(x, W)
    return (logits_t.T, 2)
